# Initial kernel scaffold; baseline (speedup 1.0000x reference)
import jax
import jax.numpy as jnp
from jax.experimental import pallas as pl


def kernel(train_inputs, train_types, node_neigh, node_embeddings, node_type_embeddings, trans_weights, trans_weights_s1, trans_weights_s2):
    raise NotImplementedError("write your pallas kernel here")


# speedup vs baseline: 73.9420x; 73.9420x over previous
import functools

import jax
import jax.numpy as jnp
from jax import lax
from jax.experimental import pallas as pl
from jax.experimental.pallas import tpu as pltpu
from jax.experimental.pallas import tpu_sc as plsc

NUM_NODES = 100000
EMB = 128
EMB_U = 16
T = 4
DIM_A = 20
B = 16384
NEIGH = 10

NTILES = 32
BT = B // NTILES
SUB = 8
BSUB = BT // SUB
RSUB = BSUB * T * NEIGH
CHUNK = 128
NCH = RSUB // CHUNK
NE_CH = 128
NE_NCH = BT // NE_CH


NSUB_T = BSUB * NEIGH


def _sc_body(neigh_hbm, ti_hbm, nte_tab, ne_tab, tpat_hbm, tab_p, nte_out, ne_out,
             idx_raw, idx2, rows, ti_idx, ne_rows, pat_v, sem, sem2):
    pltpu.sync_copy(tab_p.at[0, 0, pl.ds(0, 16)], rows.at[0])
    cid = lax.axis_index("c")
    sid = lax.axis_index("s")
    wid = cid * 16 + sid

    pltpu.sync_copy(tpat_hbm, pat_v)

    for sub in range(SUB):
        b0 = wid * BT + sub * BSUB
        nbase = pl.multiple_of(b0 * (T * NEIGH), RSUB)
        pltpu.sync_copy(neigh_hbm.at[pl.ds(nbase, RSUB)], idx_raw)

        def idxbody(j, _):
            o = j * 16
            v = idx_raw[pl.ds(o, 16)]
            idx2[pl.ds(o, 16)] = v * T + pat_v[pl.ds(o, 16)]
            return 0

        lax.fori_loop(0, RSUB // 16, idxbody, 0)

        cps = [
            pltpu.async_copy(
                nte_tab.at[idx2.at[pl.ds(c * CHUNK, CHUNK)]],
                rows.at[pl.ds(c * CHUNK, CHUNK)],
                sem,
            )
            for c in range(NCH)
        ]
        for cp in cps:
            cp.wait()
        pltpu.sync_copy(rows, nte_out.at[pl.ds(nbase, RSUB)])

    tbase = pl.multiple_of(wid * BT, BT)
    pltpu.sync_copy(ti_hbm.at[pl.ds(tbase, BT)], ti_idx)
    for c in range(NE_NCH):
        pltpu.async_copy(
            ne_tab.at[ti_idx.at[pl.ds(c * NE_CH, NE_CH)]], ne_rows, sem2
        ).wait()
        nbase2 = pl.multiple_of(wid * BT + c * NE_CH, NE_CH)
        pltpu.sync_copy(ne_rows, ne_out.at[pl.ds(nbase2, NE_CH)])


@functools.cache
def _sc_gather_fn():
    return functools.partial(
        pl.kernel,
        out_type=[
            jax.ShapeDtypeStruct((B * T * NEIGH, EMB_U), jnp.float32),
            jax.ShapeDtypeStruct((B, EMB), jnp.float32),
        ],
        mesh=plsc.VectorSubcoreMesh(core_axis_name="c", subcore_axis_name="s"),
        compiler_params=pltpu.CompilerParams(use_tc_tiling_on_sc=False),
        scratch_types=[
            pltpu.VMEM((RSUB,), jnp.int32),
            pltpu.VMEM((RSUB,), jnp.int32),
            pltpu.VMEM((RSUB, EMB_U), jnp.float32),
            pltpu.VMEM((BT,), jnp.int32),
            pltpu.VMEM((NE_CH, EMB), jnp.float32),
            pltpu.VMEM((RSUB,), jnp.int32),
            pltpu.SemaphoreType.DMA,
            pltpu.SemaphoreType.DMA,
        ],
    )(_sc_body)


def _tc_body(ntr_ref, ne_ref, oh_ref, segm_ref, s1b_ref, s2b_ref, selm_ref,
             exp4_ref, tile_ref, fold_ref, w_ref, out_ref):
    f32 = jnp.float32
    nte = jnp.dot(ntr_ref[...], segm_ref[...], preferred_element_type=f32)
    oh = oh_ref[...]
    z = jnp.tanh(jnp.dot(nte, s1b_ref[...], preferred_element_type=f32))
    sall = jnp.dot(z, s2b_ref[...], preferred_element_type=f32)
    oh4 = jnp.concatenate([oh, oh, oh, oh], axis=1)
    scores = jnp.dot(sall * oh4, selm_ref[...], preferred_element_type=f32)
    m = jnp.max(scores, axis=1, keepdims=True)
    e = jnp.exp(scores - m)
    att = e / jnp.sum(e, axis=1, keepdims=True)
    att_exp = jnp.dot(att, exp4_ref[...], preferred_element_type=f32)
    att_emb = jnp.dot(nte * att_exp, fold_ref[...], preferred_element_type=f32)
    oh_exp = jnp.dot(oh, exp4_ref[...], preferred_element_type=f32)
    att_rep = jnp.dot(att_emb, tile_ref[...], preferred_element_type=f32)
    x = oh_exp * att_rep
    out = ne_ref[...] + jnp.dot(x, w_ref[...], preferred_element_type=f32)
    nrm = jnp.maximum(jnp.sqrt(jnp.sum(out * out, axis=1, keepdims=True)), 1e-12)
    out_ref[...] = out / nrm


def _tc_dense(ntr, ne, oh, segm, s1b, s2b, selm, exp4, tile_m, fold, wcat):
    bs = 2048
    grid = (B // bs,)
    full = lambda a: pl.BlockSpec(a.shape, lambda i: (0,) * a.ndim)
    return pl.pallas_call(
        _tc_body,
        grid=grid,
        in_specs=[
            pl.BlockSpec((bs, T * NEIGH * EMB_U), lambda i: (i, 0)),
            pl.BlockSpec((bs, EMB), lambda i: (i, 0)),
            pl.BlockSpec((bs, T), lambda i: (i, 0)),
            full(segm), full(s1b), full(s2b), full(selm), full(exp4),
            full(tile_m), full(fold), full(wcat),
        ],
        out_specs=pl.BlockSpec((bs, EMB), lambda i: (i, 0)),
        out_shape=jax.ShapeDtypeStruct((B, EMB), jnp.float32),
    )(ntr, ne, oh, segm, s1b, s2b, selm, exp4, tile_m, fold, wcat)


def kernel(train_inputs, train_types, node_neigh, node_embeddings,
           node_type_embeddings, trans_weights, trans_weights_s1,
           trans_weights_s2):
    f32 = jnp.float32
    neigh_flat = node_neigh.astype(jnp.int32).reshape(B * T * NEIGH)
    ti = train_inputs.astype(jnp.int32)
    nte_tab = node_type_embeddings.reshape(NUM_NODES * T, EMB_U)
    tpat = ((jnp.arange(RSUB, dtype=jnp.int32) % (T * NEIGH)) // NEIGH)

    tab_p = node_type_embeddings.transpose(1, 2, 0)
    ntr, ne = _sc_gather_fn()(neigh_flat, ti, nte_tab, node_embeddings, tpat, tab_p)
    ntr_flat = ntr.reshape(B, T * NEIGH * EMB_U)

    oh = (train_types[:, None] == jnp.arange(T)[None, :]).astype(f32)

    eyeT = jnp.eye(T, dtype=f32)
    eyeU = jnp.eye(EMB_U, dtype=f32)
    segm = jnp.kron(eyeT, jnp.kron(jnp.ones((NEIGH, 1), f32), eyeU))
    s1cat = trans_weights_s1.transpose(1, 0, 2).reshape(EMB_U, T * DIM_A)
    s1b = jnp.kron(eyeT, s1cat)
    s2sq = trans_weights_s2[:, :, 0]
    s2small = (eyeT[:, None, :] * s2sq[:, :, None]).reshape(T * DIM_A, T)
    s2b = jnp.kron(eyeT, s2small)
    selm = jnp.kron(eyeT, jnp.ones((T, 1), f32))
    exp4 = jnp.kron(eyeT, jnp.ones((1, EMB_U), f32))
    tile_m = jnp.kron(jnp.ones((1, T), f32), eyeU)
    fold = jnp.kron(jnp.ones((T, 1), f32), eyeU)
    wcat = trans_weights.reshape(T * EMB_U, EMB)

    return _tc_dense(ntr_flat, ne, oh, segm, s1b, s2b, selm, exp4, tile_m, fold, wcat)
